# Initial kernel scaffold; baseline (speedup 1.0000x reference)
import jax
import jax.numpy as jnp
from jax.experimental import pallas as pl


def kernel(type_bits, pattern_table_1, pattern_table_2, pattern_table_3, pattern_table_4, position_table_1, position_table_2, position_table_3, position_table_4, conn4):
    raise NotImplementedError("write your pallas kernel here")


# speedup vs baseline: 20.7788x; 20.7788x over previous
import functools

import jax
import jax.numpy as jnp
from jax import lax
from jax.experimental import pallas as pl
from jax.experimental.pallas import tpu as pltpu
from jax.experimental.pallas import tpu_sc as plsc

_B = 262144
_NC, _NS, _L = 2, 16, 16
_NW = _NC * _NS
_EPW = _B // _NW
_CH = 1024
_NSUB = _EPW // _CH

_PT_SIZE = (8, 64, 512, 4096)
_POS_SIZE = (64, 512, 4096, 4096)
_PT_OFF = []
_POS_OFF = []
_off = 0
for _n in range(4):
    _PT_OFF.append(_off)
    _off += 3 * _PT_SIZE[_n]
for _n in range(4):
    _POS_OFF.append(_off)
    _off += 5 * _POS_SIZE[_n]
_TABF_LEN = _off
_H_OFF = 5 * 4096
_TABI_LEN = _H_OFF + 5 * 8


@functools.cache
def _build_sc_forward():
    mesh = plsc.VectorSubcoreMesh(
        core_axis_name="c", subcore_axis_name="s",
        num_cores=_NC, num_subcores=_NS)
    return pl.kernel(
        _sc_body,
        out_type=jax.ShapeDtypeStruct((_B * 32,), jnp.float32),
        mesh=mesh,
        scratch_types=[
            pltpu.VMEM((_TABF_LEN,), jnp.float32),
            pltpu.VMEM((_TABI_LEN,), jnp.int32),
            pltpu.VMEM((_CH * 12,), jnp.int32),
            pltpu.VMEM((_CH * 32,), jnp.float32),
        ],
        compiler_params=pltpu.CompilerParams(needs_layout_passes=False),
    )


def _sc_body(tb_hbm, tabf_hbm, tabi_hbm, out_hbm, tabf, tabi, bitsv, outv):
    wid = lax.axis_index("s") * _NC + lax.axis_index("c")
    pltpu.sync_copy(tabf_hbm, tabf)
    pltpu.sync_copy(tabi_hbm, tabi)
    lane = lax.iota(jnp.int32, _L)
    lane12 = lane * 12
    lane32 = lane * 32

    for s in range(_NSUB):
        eb = wid * _EPW + s * _CH
        pltpu.sync_copy(tb_hbm.at[pl.ds(pl.multiple_of(eb * 12, 8), _CH * 12)],
                        bitsv)

        def vec_body(v, carry):
            bbase = v * (_L * 12) + lane12
            addr = plsc.load_gather(bitsv, [bbase])
            for k in range(1, 12):
                bit = plsc.load_gather(bitsv, [bbase + k])
                addr = addr * 2 + bit
            ov = v * (_L * 32) + lane32
            for n in range(4):
                size = _PT_SIZE[n]
                an = jnp.bitwise_and(addr, size - 1) if n < 3 else addr
                pt_base = _PT_OFF[n] + an
                t0 = plsc.load_gather(tabf, [pt_base])
                t1 = plsc.load_gather(tabf, [pt_base + size])
                t2 = plsc.load_gather(tabf, [pt_base + 2 * size])
                h0 = (t0 > 0.5).astype(jnp.int32)
                h1 = (t1 > 0.5).astype(jnp.int32)
                h2 = (t2 > 0.5).astype(jnp.int32)
                hard = h0 * 4 + h1 * 2 + h2
                plsc.store_scatter(outv, [ov + (n * 8 + 0)], t0)
                plsc.store_scatter(outv, [ov + (n * 8 + 1)], t1)
                plsc.store_scatter(outv, [ov + (n * 8 + 2)], t2)
                if n < 3:
                    ap = _POS_OFF[n] + an * 8 + hard
                    psize = _POS_SIZE[n]
                    for j in range(5):
                        pj = plsc.load_gather(tabf, [ap + j * psize])
                        plsc.store_scatter(outv, [ov + (n * 8 + 3 + j)], pj)
                else:
                    for j in range(5):
                        av = plsc.load_gather(tabi, [an + j * 4096])
                        hv = plsc.load_gather(tabi, [_H_OFF + j * 8 + hard])
                        pj = plsc.load_gather(tabf, [_POS_OFF[3] + av + hv])
                        plsc.store_scatter(outv, [ov + (n * 8 + 3 + j)], pj)
            return carry

        lax.fori_loop(0, _CH // _L, vec_body, 0)
        pltpu.sync_copy(outv,
                        out_hbm.at[pl.ds(pl.multiple_of(eb * 32, 8), _CH * 32)])


def _aux_tables(conn4):
    c = conn4.astype(jnp.int32)
    w = (jnp.int32(1) << (11 - jnp.arange(12, dtype=jnp.int32)))
    ai = jnp.arange(4096, dtype=jnp.int32)
    sa = jnp.clip(11 - c, 0, 31)
    bits_a = (ai[None, None, :] >> sa[:, :, None]) & 1
    a_tab = jnp.sum(
        jnp.where((c < 12)[:, :, None], bits_a, 0) * w[None, :, None], axis=1)
    a_tab = a_tab.astype(jnp.int32) + (jnp.arange(5, dtype=jnp.int32) * 4096)[:, None]
    hi = jnp.arange(8, dtype=jnp.int32)
    sh = jnp.clip(14 - c, 0, 31)
    bits_h = (hi[None, None, :] >> sh[:, :, None]) & 1
    h_tab = jnp.sum(
        jnp.where((c >= 12)[:, :, None], bits_h, 0) * w[None, :, None],
        axis=1).astype(jnp.int32)
    return a_tab, h_tab


def kernel(type_bits, pattern_table_1, pattern_table_2, pattern_table_3,
           pattern_table_4, position_table_1, position_table_2,
           position_table_3, position_table_4, conn4):
    assert type_bits.shape == (_B, 12)
    pts = [pattern_table_1, pattern_table_2, pattern_table_3, pattern_table_4]
    poss = [position_table_1, position_table_2, position_table_3,
            position_table_4]
    tabf = jnp.concatenate([p.reshape(-1) for p in pts]
                           + [p.reshape(-1) for p in poss])
    a_tab, h_tab = _aux_tables(conn4)
    tabi = jnp.concatenate([a_tab.reshape(-1), h_tab.reshape(-1)])
    out = _build_sc_forward()(type_bits.reshape(-1), tabf, tabi)
    return out.reshape(_B, 4, 8)


# speedup vs baseline: 132.2498x; 6.3646x over previous
import functools

import jax
import jax.numpy as jnp
from jax import lax
from jax.experimental import pallas as pl
from jax.experimental.pallas import tpu as pltpu
from jax.experimental.pallas import tpu_sc as plsc

_B = 262144
_NC, _NS, _L = 2, 16, 16
_NW = _NC * _NS
_EPW = _B // _NW
_CH = 1024
_NSUB = _EPW // _CH
_NBT = _B // 128

_PT_SIZE = (8, 64, 512, 4096)
_POS_SIZE = (64, 512, 4096, 4096)
_PT_OFF = []
_POS_OFF = []
_off = 0
for _n in range(4):
    _PT_OFF.append(_off)
    _off += 3 * _PT_SIZE[_n]
for _n in range(4):
    _POS_OFF.append(_off)
    _off += 5 * _POS_SIZE[_n]
_TABF_LEN = _off
_H_OFF = 5 * 4096
_TABI_LEN = _H_OFF + 5 * 8


def _sc_body(tb_hbm, tabf_hbm, tabi_hbm, out_hbm, tabf, tabi, bitsv, outv,
             sem_in, sem_out):
    wid = lax.axis_index("s") * _NC + lax.axis_index("c")
    pltpu.sync_copy(tabf_hbm, tabf)
    pltpu.sync_copy(tabi_hbm, tabi)

    for s in range(_NSUB):
        base = wid * _EPW + s * _CH
        handles = []
        for k in range(12):
            src = tb_hbm.at[pl.ds(pl.multiple_of(k * _B + base, 8), _CH)]
            dst = bitsv.at[pl.ds(k * _CH, _CH)]
            handles.append(pltpu.async_copy(src, dst, sem_in))
        for h in handles:
            h.wait()

        def vec_body(v, carry):
            e = v * _L
            addr = bitsv[pl.ds(e, _L)]
            for k in range(1, 12):
                addr = addr * 2 + bitsv[pl.ds(k * _CH + e, _L)]
            ob = (v // 8) * 1024 + (v % 8) * _L
            for n in range(4):
                size = _PT_SIZE[n]
                an = jnp.bitwise_and(addr, size - 1) if n < 3 else addr
                pt_base = _PT_OFF[n] + an
                t0 = plsc.load_gather(tabf, [pt_base])
                t1 = plsc.load_gather(tabf, [pt_base + size])
                t2 = plsc.load_gather(tabf, [pt_base + 2 * size])
                h0 = (t0 > 0.5).astype(jnp.int32)
                h1 = (t1 > 0.5).astype(jnp.int32)
                h2 = (t2 > 0.5).astype(jnp.int32)
                hard = h0 * 4 + h1 * 2 + h2
                nb = n * (8 * _CH) + ob
                outv[pl.ds(nb, _L)] = t0
                outv[pl.ds(nb + 128, _L)] = t1
                outv[pl.ds(nb + 256, _L)] = t2
                if n < 3:
                    ap = _POS_OFF[n] + an * 8 + hard
                    psize = _POS_SIZE[n]
                    for j in range(5):
                        pj = plsc.load_gather(tabf, [ap + j * psize])
                        outv[pl.ds(nb + (3 + j) * 128, _L)] = pj
                else:
                    for j in range(5):
                        av = plsc.load_gather(tabi, [an + j * 4096])
                        hv = plsc.load_gather(tabi, [_H_OFF + j * 8 + hard])
                        pj = plsc.load_gather(tabf, [_POS_OFF[3] + av + hv])
                        outv[pl.ds(nb + (3 + j) * 128, _L)] = pj
            return carry

        lax.fori_loop(0, _CH // _L, vec_body, 0)

        tb0 = base // 128
        handles = []
        for n in range(4):
            src = outv.at[pl.ds(n * (8 * _CH), 8 * _CH)]
            dst = out_hbm.at[pl.ds(
                pl.multiple_of(n * (_NBT * 1024) + tb0 * 1024, 8), 8 * _CH)]
            handles.append(pltpu.async_copy(src, dst, sem_out))
        for h in handles:
            h.wait()


@functools.cache
def _build_sc_forward():
    mesh = plsc.VectorSubcoreMesh(
        core_axis_name="c", subcore_axis_name="s",
        num_cores=_NC, num_subcores=_NS)
    return pl.kernel(
        _sc_body,
        out_type=jax.ShapeDtypeStruct((4 * _NBT * 1024,), jnp.float32),
        mesh=mesh,
        scratch_types=[
            pltpu.VMEM((_TABF_LEN,), jnp.float32),
            pltpu.VMEM((_TABI_LEN,), jnp.int32),
            pltpu.VMEM((_CH * 12,), jnp.int32),
            pltpu.VMEM((_CH * 32,), jnp.float32),
            pltpu.SemaphoreType.DMA,
            pltpu.SemaphoreType.DMA,
        ],
        compiler_params=pltpu.CompilerParams(needs_layout_passes=False),
    )


def _aux_tables(conn4):
    c = conn4.astype(jnp.int32)
    w = (jnp.int32(1) << (11 - jnp.arange(12, dtype=jnp.int32)))
    ai = jnp.arange(4096, dtype=jnp.int32)
    sa = jnp.clip(11 - c, 0, 31)
    bits_a = (ai[None, None, :] >> sa[:, :, None]) & 1
    a_tab = jnp.sum(
        jnp.where((c < 12)[:, :, None], bits_a, 0) * w[None, :, None], axis=1)
    a_tab = a_tab.astype(jnp.int32) + (jnp.arange(5, dtype=jnp.int32) * 4096)[:, None]
    hi = jnp.arange(8, dtype=jnp.int32)
    sh = jnp.clip(14 - c, 0, 31)
    bits_h = (hi[None, None, :] >> sh[:, :, None]) & 1
    h_tab = jnp.sum(
        jnp.where((c >= 12)[:, :, None], bits_h, 0) * w[None, :, None],
        axis=1).astype(jnp.int32)
    return a_tab, h_tab


def kernel(type_bits, pattern_table_1, pattern_table_2, pattern_table_3,
           pattern_table_4, position_table_1, position_table_2,
           position_table_3, position_table_4, conn4):
    assert type_bits.shape == (_B, 12)
    pts = [pattern_table_1, pattern_table_2, pattern_table_3, pattern_table_4]
    poss = [position_table_1, position_table_2, position_table_3,
            position_table_4]
    tabf = jnp.concatenate([p.reshape(-1) for p in pts]
                           + [p.reshape(-1) for p in poss])
    a_tab, h_tab = _aux_tables(conn4)
    tabi = jnp.concatenate([a_tab.reshape(-1), h_tab.reshape(-1)])
    tbp = jnp.transpose(type_bits).reshape(-1)
    out = _build_sc_forward()(tbp, tabf, tabi)
    return (out.reshape(4, _NBT, 8, 128)
            .transpose(1, 3, 0, 2).reshape(_B, 4, 8))


# speedup vs baseline: 198.8389x; 1.5035x over previous
import functools

import jax
import jax.numpy as jnp
from jax import lax
from jax.experimental import pallas as pl
from jax.experimental.pallas import tpu as pltpu
from jax.experimental.pallas import tpu_sc as plsc

_B = 262144
_NC, _NS, _L = 2, 16, 16
_NW = _NC * _NS
_EPW = _B // _NW
_CH = 1024
_NSUB = _EPW // _CH
_NBT = _B // 128

_PT_SIZE = (8, 64, 512, 4096)
_POS_SIZE = (64, 512, 4096, 4096)
_PT_OFF = []
_POS_OFF = []
_off = 0
for _n in range(4):
    _PT_OFF.append(_off)
    _off += 3 * _PT_SIZE[_n]
for _n in range(4):
    _POS_OFF.append(_off)
    _off += 5 * _POS_SIZE[_n]
_TABF_LEN = _off
_H_OFF = 5 * 4096
_TABI_LEN = _H_OFF + 5 * 8


def _sc_body(tb_hbm, tabf_hbm, tabi_hbm, out_hbm, tabf, tabi, bitsv, outv,
             sem_in, sem_out):
    wid = lax.axis_index("s") * _NC + lax.axis_index("c")
    pltpu.sync_copy(tabf_hbm, tabf)
    pltpu.sync_copy(tabi_hbm, tabi)

    for s in range(_NSUB):
        base = wid * _EPW + s * _CH
        handles = []
        for k in range(12):
            src = tb_hbm.at[pl.ds(pl.multiple_of(k * _B + base, 8), _CH)]
            dst = bitsv.at[pl.ds(k * _CH, _CH)]
            handles.append(pltpu.async_copy(src, dst, sem_in))
        for h in handles:
            h.wait()

        @plsc.parallel_loop(0, _CH // _L, 1, unroll=4)
        def vec_body(v):
            e = v * _L
            bs = [bitsv[pl.ds(k * _CH + e, _L)] for k in range(12)]
            pairs = [bs[k] * 2 + bs[k + 1] for k in range(0, 12, 2)]
            quads = [pairs[i] * 4 + pairs[i + 1] for i in range(0, 6, 2)]
            addr = (quads[0] * 16 + quads[1]) * 16 + quads[2]
            ob = (v // 8) * 1024 + (v % 8) * _L
            for n in range(4):
                size = _PT_SIZE[n]
                an = jnp.bitwise_and(addr, size - 1) if n < 3 else addr
                pt_base = _PT_OFF[n] + an
                t0 = plsc.load_gather(tabf, [pt_base])
                t1 = plsc.load_gather(tabf, [pt_base + size])
                t2 = plsc.load_gather(tabf, [pt_base + 2 * size])
                h0 = (t0 > 0.5).astype(jnp.int32)
                h1 = (t1 > 0.5).astype(jnp.int32)
                h2 = (t2 > 0.5).astype(jnp.int32)
                hard = h0 * 4 + h1 * 2 + h2
                nb = n * (8 * _CH) + ob
                outv[pl.ds(nb, _L)] = t0
                outv[pl.ds(nb + 128, _L)] = t1
                outv[pl.ds(nb + 256, _L)] = t2
                if n < 3:
                    ap = _POS_OFF[n] + an * 8 + hard
                    psize = _POS_SIZE[n]
                    for j in range(5):
                        pj = plsc.load_gather(tabf, [ap + j * psize])
                        outv[pl.ds(nb + (3 + j) * 128, _L)] = pj
                else:
                    for j in range(5):
                        av = plsc.load_gather(tabi, [an + j * 4096])
                        hv = plsc.load_gather(tabi, [_H_OFF + j * 8 + hard])
                        pj = plsc.load_gather(tabf, [_POS_OFF[3] + av + hv])
                        outv[pl.ds(nb + (3 + j) * 128, _L)] = pj

        tb0 = base // 128
        handles = []
        for n in range(4):
            src = outv.at[pl.ds(n * (8 * _CH), 8 * _CH)]
            dst = out_hbm.at[pl.ds(
                pl.multiple_of(n * (_NBT * 1024) + tb0 * 1024, 8), 8 * _CH)]
            handles.append(pltpu.async_copy(src, dst, sem_out))
        for h in handles:
            h.wait()


@functools.cache
def _build_sc_forward():
    mesh = plsc.VectorSubcoreMesh(
        core_axis_name="c", subcore_axis_name="s",
        num_cores=_NC, num_subcores=_NS)
    return pl.kernel(
        _sc_body,
        out_type=jax.ShapeDtypeStruct((4 * _NBT * 1024,), jnp.float32),
        mesh=mesh,
        scratch_types=[
            pltpu.VMEM((_TABF_LEN,), jnp.float32),
            pltpu.VMEM((_TABI_LEN,), jnp.int32),
            pltpu.VMEM((_CH * 12,), jnp.int32),
            pltpu.VMEM((_CH * 32,), jnp.float32),
            pltpu.SemaphoreType.DMA,
            pltpu.SemaphoreType.DMA,
        ],
        compiler_params=pltpu.CompilerParams(needs_layout_passes=False),
    )


def _aux_tables(conn4):
    c = conn4.astype(jnp.int32)
    w = (jnp.int32(1) << (11 - jnp.arange(12, dtype=jnp.int32)))
    ai = jnp.arange(4096, dtype=jnp.int32)
    sa = jnp.clip(11 - c, 0, 31)
    bits_a = (ai[None, None, :] >> sa[:, :, None]) & 1
    a_tab = jnp.sum(
        jnp.where((c < 12)[:, :, None], bits_a, 0) * w[None, :, None], axis=1)
    a_tab = a_tab.astype(jnp.int32) + (jnp.arange(5, dtype=jnp.int32) * 4096)[:, None]
    hi = jnp.arange(8, dtype=jnp.int32)
    sh = jnp.clip(14 - c, 0, 31)
    bits_h = (hi[None, None, :] >> sh[:, :, None]) & 1
    h_tab = jnp.sum(
        jnp.where((c >= 12)[:, :, None], bits_h, 0) * w[None, :, None],
        axis=1).astype(jnp.int32)
    return a_tab, h_tab


def kernel(type_bits, pattern_table_1, pattern_table_2, pattern_table_3,
           pattern_table_4, position_table_1, position_table_2,
           position_table_3, position_table_4, conn4):
    assert type_bits.shape == (_B, 12)
    pts = [pattern_table_1, pattern_table_2, pattern_table_3, pattern_table_4]
    poss = [position_table_1, position_table_2, position_table_3,
            position_table_4]
    tabf = jnp.concatenate([p.reshape(-1) for p in pts]
                           + [p.reshape(-1) for p in poss])
    a_tab, h_tab = _aux_tables(conn4)
    tabi = jnp.concatenate([a_tab.reshape(-1), h_tab.reshape(-1)])
    tbp = jnp.transpose(type_bits).reshape(-1)
    out = _build_sc_forward()(tbp, tabf, tabi)
    return (out.reshape(4, _NBT, 8, 128)
            .transpose(1, 3, 0, 2).reshape(_B, 4, 8))
